# baseline (speedup 1.0000x reference)
import functools

import jax
import jax.numpy as jnp
from jax import lax
from jax.experimental import pallas as pl
from jax.experimental.pallas import tpu as pltpu
from jax.experimental.pallas import tpu_sc as plsc

B, S, K = 2, 2048, 8
D_MODEL, D_FF = 1024, 4096
N_NEURONS, N_PATTERNS, K_PATTERNS = 512, 64, 8
T = B * S

NC, NS = 2, 16
NW = NC * NS
CT = T // NW
NG = CT // 16

TOK_BLK = 1024
N_BLK = T // TOK_BLK
FF_CHUNKS = 2


def _transpose_pa(pattern_affinity):
  def body(pa_ref, out_ref):
    out_ref[...] = pa_ref[...].T

  return pl.pallas_call(
      body,
      out_shape=jax.ShapeDtypeStruct((N_NEURONS, N_PATTERNS), jnp.float32),
  )(pattern_affinity)


def _sc_scores(pa_t, idx_flat, w_flat):
  mesh = plsc.VectorSubcoreMesh(core_axis_name="c", subcore_axis_name="s")

  @functools.partial(
      pl.kernel,
      mesh=mesh,
      compiler_params=pltpu.CompilerParams(needs_layout_passes=False),
      out_type=jax.ShapeDtypeStruct((T, N_PATTERNS), jnp.float32),
      scratch_types=[
          pltpu.VMEM((N_NEURONS * N_PATTERNS,), jnp.float32),
          pltpu.VMEM((CT * K,), jnp.int32),
          pltpu.VMEM((CT * K,), jnp.float32),
          pltpu.VMEM((CT, N_PATTERNS), jnp.float32),
      ],
  )
  def sc_kernel(pa_hbm, idx_hbm, w_hbm, out_hbm, pa_v, idx_v, w_v, sc_v):
    wid = lax.axis_index("s") * NC + lax.axis_index("c")
    pltpu.sync_copy(pa_hbm, pa_v)
    pltpu.sync_copy(idx_hbm.at[pl.ds(wid * (CT * K), CT * K)], idx_v)
    pltpu.sync_copy(w_hbm.at[pl.ds(wid * (CT * K), CT * K)], w_v)
    nj = N_PATTERNS // 16

    def pair_body(t2, _):
      iv = idx_v[pl.ds(t2 * 16, 16)] * N_PATTERNS
      wv = w_v[pl.ds(t2 * 16, 16)]
      for tt in range(2):
        accs = None
        for k in range(K):
          a = iv[tt * K + k]
          wk = wv[tt * K + k]
          rows = [wk * pa_v[pl.ds(a + j * 16, 16)] for j in range(nj)]
          accs = rows if accs is None else [x + y for x, y in zip(accs, rows)]
        for j in range(nj):
          sc_v[t2 * 2 + tt, pl.ds(j * 16, 16)] = accs[j]
      return 0

    lax.fori_loop(0, CT // 2, pair_body, 0)
    pltpu.sync_copy(sc_v, out_hbm.at[pl.ds(wid * CT, CT)])

  return sc_kernel(pa_t.reshape(-1), idx_flat, w_flat)


def _tc_body(sc_ref, x_ref, gates_ref, upw_ref, upb_ref, dnw_ref, dnb_ref,
             out_ref):
  s = sc_ref[...]
  work = s
  m0 = jnp.max(work, axis=1, keepdims=True)
  th = m0
  for _ in range(K_PATTERNS - 1):
    work = jnp.where(work >= th, -jnp.inf, work)
    th = jnp.max(work, axis=1, keepdims=True)
  e = jnp.where(s >= th, jnp.exp(s - m0), 0.0)
  pw = (e / jnp.sum(e, axis=1, keepdims=True)).astype(jnp.bfloat16)

  ffn_gate = jnp.dot(pw, gates_ref[...], preferred_element_type=jnp.float32)
  h = lax.dot_general(x_ref[...], upw_ref[...], (((1,), (1,)), ((), ())),
                      preferred_element_type=jnp.float32)
  h = h + upb_ref[...]
  h = h * jax.nn.sigmoid(ffn_gate)
  h = 0.5 * h * (1.0 + lax.erf(h * (1.0 / jnp.sqrt(2.0).astype(jnp.float32))))
  out = lax.dot_general(h.astype(jnp.bfloat16), dnw_ref[...], (((1,), (1,)), ((), ())),
                        preferred_element_type=jnp.float32)
  out_ref[...] = out + dnb_ref[...]


def _tc_ffn(scores2, x2, gates, up_w_t, up_b2, down_w_t, down_b2):
  return pl.pallas_call(
      _tc_body,
      grid=(N_BLK,),
      in_specs=[
          pl.BlockSpec((TOK_BLK, N_PATTERNS), lambda i: (i, 0)),
          pl.BlockSpec((TOK_BLK, D_MODEL), lambda i: (i, 0)),
          pl.BlockSpec((N_PATTERNS, D_FF), lambda i: (0, 0)),
          pl.BlockSpec((D_FF, D_MODEL), lambda i: (0, 0)),
          pl.BlockSpec((1, D_FF), lambda i: (0, 0)),
          pl.BlockSpec((D_MODEL, D_FF), lambda i: (0, 0)),
          pl.BlockSpec((1, D_MODEL), lambda i: (0, 0)),
      ],
      out_specs=pl.BlockSpec((TOK_BLK, D_MODEL), lambda i: (i, 0)),
      out_shape=jax.ShapeDtypeStruct((T, D_MODEL), jnp.float32),
  )(scores2, x2, gates, up_w_t, up_b2, down_w_t, down_b2)


def kernel(x, router_out, topk_neuron_idx, topk_neuron_weights,
           pattern_affinity, gates, up_w, up_b, down_w, down_b):
  del router_out
  x2 = x.reshape(T, D_MODEL)
  idx_flat = topk_neuron_idx.reshape(T * K).astype(jnp.int32)
  w_flat = topk_neuron_weights.reshape(T * K)
  pa_t = _transpose_pa(pattern_affinity)

  scores2 = _sc_scores(pa_t, idx_flat, w_flat)

  up_b2 = up_b.reshape(1, D_FF)
  down_b2 = down_b.reshape(1, D_MODEL)
  out2 = _tc_ffn(scores2, x2.astype(jnp.bfloat16), gates.astype(jnp.bfloat16),
                 up_w.astype(jnp.bfloat16), up_b2,
                 down_w.astype(jnp.bfloat16), down_b2)
  return out2.reshape(B, S, D_MODEL)


# speedup vs baseline: 1.0857x; 1.0857x over previous
import functools

import jax
import jax.numpy as jnp
from jax import lax
from jax.experimental import pallas as pl
from jax.experimental.pallas import tpu as pltpu
from jax.experimental.pallas import tpu_sc as plsc

B, S, K = 2, 2048, 8
D_MODEL, D_FF = 1024, 4096
N_NEURONS, N_PATTERNS, K_PATTERNS = 512, 64, 8
T = B * S

NC, NS = 2, 16
NW = NC * NS
CT = T // NW
NG = CT // 16

TOK_BLK = 512
N_BLK = T // TOK_BLK
FF_CHUNKS = 2


def _transpose_pa(pattern_affinity):
  def body(pa_ref, out_ref):
    out_ref[...] = pa_ref[...].T

  return pl.pallas_call(
      body,
      out_shape=jax.ShapeDtypeStruct((N_NEURONS, N_PATTERNS), jnp.float32),
  )(pattern_affinity)


def _sc_scores(pa_t, idx_flat, w_flat):
  mesh = plsc.VectorSubcoreMesh(core_axis_name="c", subcore_axis_name="s")

  @functools.partial(
      pl.kernel,
      mesh=mesh,
      compiler_params=pltpu.CompilerParams(needs_layout_passes=False),
      out_type=jax.ShapeDtypeStruct((T, N_PATTERNS), jnp.float32),
      scratch_types=[
          pltpu.VMEM((N_NEURONS * N_PATTERNS,), jnp.float32),
          pltpu.VMEM((CT * K,), jnp.int32),
          pltpu.VMEM((CT * K,), jnp.float32),
          pltpu.VMEM((CT, N_PATTERNS), jnp.float32),
      ],
  )
  def sc_kernel(pa_hbm, idx_hbm, w_hbm, out_hbm, pa_v, idx_v, w_v, sc_v):
    wid = lax.axis_index("s") * NC + lax.axis_index("c")
    pltpu.sync_copy(pa_hbm, pa_v)
    pltpu.sync_copy(idx_hbm.at[pl.ds(wid * (CT * K), CT * K)], idx_v)
    pltpu.sync_copy(w_hbm.at[pl.ds(wid * (CT * K), CT * K)], w_v)
    nj = N_PATTERNS // 16

    def quad_body(t4, _):
      ivs = [idx_v[pl.ds((t4 * 2 + u) * 16, 16)] * N_PATTERNS for u in range(2)]
      wvs = [w_v[pl.ds((t4 * 2 + u) * 16, 16)] for u in range(2)]
      for u in range(2):
        for tt in range(2):
          accs = None
          for k in range(K):
            a = ivs[u][tt * K + k]
            wk = wvs[u][tt * K + k]
            rows = [wk * pa_v[pl.ds(a + j * 16, 16)] for j in range(nj)]
            accs = rows if accs is None else [x + y for x, y in zip(accs, rows)]
          for j in range(nj):
            sc_v[t4 * 4 + u * 2 + tt, pl.ds(j * 16, 16)] = accs[j]
      return 0

    lax.fori_loop(0, CT // 4, quad_body, 0)
    pltpu.sync_copy(sc_v, out_hbm.at[pl.ds(wid * CT, CT)])

  return sc_kernel(pa_t.reshape(-1), idx_flat, w_flat)


def _tc_body(sc_ref, x_ref, gates_ref, upw_ref, upb_ref, dnw_ref, dnb_ref,
             out_ref):
  s = sc_ref[...]
  work = s
  m0 = jnp.max(work, axis=1, keepdims=True)
  th = m0
  for _ in range(K_PATTERNS - 1):
    work = jnp.where(work >= th, -jnp.inf, work)
    th = jnp.max(work, axis=1, keepdims=True)
  e = jnp.where(s >= th, jnp.exp(s - m0), 0.0)
  pw = e / jnp.sum(e, axis=1, keepdims=True)

  ffn_gate = jnp.dot(pw, gates_ref[...], preferred_element_type=jnp.float32)
  h = lax.dot_general(x_ref[...], upw_ref[...], (((1,), (1,)), ((), ())),
                      preferred_element_type=jnp.float32)
  h = h + upb_ref[...]
  h = h * jax.nn.sigmoid(ffn_gate)
  h = 0.5 * h * (1.0 + lax.erf(h * (1.0 / jnp.sqrt(2.0).astype(jnp.float32))))
  out = lax.dot_general(h, dnw_ref[...], (((1,), (1,)), ((), ())),
                        preferred_element_type=jnp.float32)
  out_ref[...] = out + dnb_ref[...]


def _tc_ffn(scores2, x2, gates, up_w_t, up_b2, down_w_t, down_b2):
  return pl.pallas_call(
      _tc_body,
      grid=(N_BLK,),
      in_specs=[
          pl.BlockSpec((TOK_BLK, N_PATTERNS), lambda i: (i, 0)),
          pl.BlockSpec((TOK_BLK, D_MODEL), lambda i: (i, 0)),
          pl.BlockSpec((N_PATTERNS, D_FF), lambda i: (0, 0)),
          pl.BlockSpec((D_FF, D_MODEL), lambda i: (0, 0)),
          pl.BlockSpec((1, D_FF), lambda i: (0, 0)),
          pl.BlockSpec((D_MODEL, D_FF), lambda i: (0, 0)),
          pl.BlockSpec((1, D_MODEL), lambda i: (0, 0)),
      ],
      out_specs=pl.BlockSpec((TOK_BLK, D_MODEL), lambda i: (i, 0)),
      out_shape=jax.ShapeDtypeStruct((T, D_MODEL), jnp.float32),
  )(scores2, x2, gates, up_w_t, up_b2, down_w_t, down_b2)


def kernel(x, router_out, topk_neuron_idx, topk_neuron_weights,
           pattern_affinity, gates, up_w, up_b, down_w, down_b):
  del router_out
  x2 = x.reshape(T, D_MODEL)
  idx_flat = topk_neuron_idx.reshape(T * K).astype(jnp.int32)
  w_flat = topk_neuron_weights.reshape(T * K)
  pa_t = _transpose_pa(pattern_affinity)

  scores2 = _sc_scores(pa_t, idx_flat, w_flat)

  up_b2 = up_b.reshape(1, D_FF)
  down_b2 = down_b.reshape(1, D_MODEL)
  out2 = _tc_ffn(scores2, x2, gates, up_w, up_b2, down_w, down_b2)
  return out2.reshape(B, S, D_MODEL)
